# baseline (speedup 1.0000x reference)
import functools

import jax
import jax.numpy as jnp
from jax import lax
from jax.experimental import pallas as pl
from jax.experimental.pallas import tpu as pltpu
from jax.experimental.pallas import tpu_sc as plsc

VOCAB = 100000
EMBED_DIM = 64
HIDDEN_DIM = 128
BATCH = 1024

NC, NS = 2, 16
NW = NC * NS
B_PER_W = BATCH // NW


def _sc_gather(table, idx):
    mesh = plsc.VectorSubcoreMesh(core_axis_name="c", subcore_axis_name="s")

    @functools.partial(
        pl.kernel,
        mesh=mesh,
        out_type=jax.ShapeDtypeStruct((BATCH, EMBED_DIM), jnp.float32),
        scratch_types=[
            pltpu.VMEM((B_PER_W,), jnp.int32),
            pltpu.VMEM((B_PER_W, EMBED_DIM), jnp.float32),
            pltpu.SemaphoreType.DMA,
        ],
        compiler_params=pltpu.CompilerParams(use_tc_tiling_on_sc=False),
    )
    def gather_kernel(table_hbm, idx_hbm, out_hbm, idx_v, rows_v, sem):
        wid = lax.axis_index("s") * NC + lax.axis_index("c")
        base = wid * B_PER_W
        pltpu.sync_copy(idx_hbm.at[pl.ds(base, B_PER_W)], idx_v)
        pltpu.async_copy(table_hbm.at[idx_v], rows_v, sem).wait()
        pltpu.sync_copy(rows_v, out_hbm.at[pl.ds(base, B_PER_W)])

    return gather_kernel(table, idx)


BLK_V = 2048


def _mlp_body(emb_ref, w1_ref, b1_ref, w2_ref, b2_ref, out_ref, hidden_ref):
    @pl.when(pl.program_id(0) == 0)
    def _():
        h = jnp.dot(emb_ref[...], w1_ref[...],
                    preferred_element_type=jnp.float32)
        hidden_ref[...] = jnp.maximum(h + b1_ref[...], 0.0)

    out_ref[...] = (
        jnp.dot(hidden_ref[...], w2_ref[...],
                preferred_element_type=jnp.float32)
        + b2_ref[...]
    )


def _tc_mlp(embedded, W1, b1, W2, b2):
    grid = (pl.cdiv(VOCAB, BLK_V),)
    return pl.pallas_call(
        _mlp_body,
        grid=grid,
        in_specs=[
            pl.BlockSpec((BATCH, EMBED_DIM), lambda i: (0, 0)),
            pl.BlockSpec((EMBED_DIM, HIDDEN_DIM), lambda i: (0, 0)),
            pl.BlockSpec((1, HIDDEN_DIM), lambda i: (0, 0)),
            pl.BlockSpec((HIDDEN_DIM, BLK_V), lambda i: (0, i)),
            pl.BlockSpec((1, BLK_V), lambda i: (0, i)),
        ],
        out_specs=pl.BlockSpec((BATCH, BLK_V), lambda i: (0, i)),
        out_shape=jax.ShapeDtypeStruct((BATCH, VOCAB), jnp.float32),
        scratch_shapes=[pltpu.VMEM((BATCH, HIDDEN_DIM), jnp.float32)],
        compiler_params=pltpu.CompilerParams(
            dimension_semantics=("arbitrary",),
        ),
    )(embedded, W1, b1.reshape(1, HIDDEN_DIM), W2, b2.reshape(1, VOCAB))


def kernel(x, emb_table, W1, b1, W2, b2):
    embedded = _sc_gather(emb_table, x.astype(jnp.int32))
    return _tc_mlp(embedded, W1, b1, W2, b2)


# speedup vs baseline: 1.0019x; 1.0019x over previous
import functools

import jax
import jax.numpy as jnp
from jax import lax
from jax.experimental import pallas as pl
from jax.experimental.pallas import tpu as pltpu
from jax.experimental.pallas import tpu_sc as plsc

VOCAB = 100000
EMBED_DIM = 64
HIDDEN_DIM = 128
BATCH = 1024

NC, NS = 2, 16
NW = NC * NS
B_PER_W = BATCH // NW


def _sc_gather(table, idx):
    mesh = plsc.VectorSubcoreMesh(core_axis_name="c", subcore_axis_name="s")

    @functools.partial(
        pl.kernel,
        mesh=mesh,
        out_type=jax.ShapeDtypeStruct((BATCH, EMBED_DIM), jnp.float32),
        scratch_types=[
            pltpu.VMEM((B_PER_W,), jnp.int32),
            pltpu.VMEM((B_PER_W, EMBED_DIM), jnp.float32),
            pltpu.SemaphoreType.DMA,
        ],
        compiler_params=pltpu.CompilerParams(use_tc_tiling_on_sc=False),
    )
    def gather_kernel(table_hbm, idx_hbm, out_hbm, idx_v, rows_v, sem):
        wid = lax.axis_index("s") * NC + lax.axis_index("c")
        base = wid * B_PER_W
        pltpu.sync_copy(idx_hbm.at[pl.ds(base, B_PER_W)], idx_v)
        pltpu.async_copy(table_hbm.at[idx_v], rows_v, sem).wait()
        pltpu.sync_copy(rows_v, out_hbm.at[pl.ds(base, B_PER_W)])

    return gather_kernel(table, idx)


BLK_V = 4096


def _mlp_body(emb_ref, w1_ref, b1_ref, w2_ref, b2_ref, out_ref, hidden_ref):
    @pl.when(pl.program_id(0) == 0)
    def _():
        h = jnp.dot(emb_ref[...], w1_ref[...],
                    preferred_element_type=jnp.float32)
        hidden_ref[...] = jnp.maximum(h + b1_ref[...], 0.0)

    out_ref[...] = (
        jnp.dot(hidden_ref[...], w2_ref[...],
                preferred_element_type=jnp.float32)
        + b2_ref[...]
    )


def _tc_mlp(embedded, W1, b1, W2, b2):
    grid = (pl.cdiv(VOCAB, BLK_V),)
    return pl.pallas_call(
        _mlp_body,
        grid=grid,
        in_specs=[
            pl.BlockSpec((BATCH, EMBED_DIM), lambda i: (0, 0)),
            pl.BlockSpec((EMBED_DIM, HIDDEN_DIM), lambda i: (0, 0)),
            pl.BlockSpec((1, HIDDEN_DIM), lambda i: (0, 0)),
            pl.BlockSpec((HIDDEN_DIM, BLK_V), lambda i: (0, i)),
            pl.BlockSpec((1, BLK_V), lambda i: (0, i)),
        ],
        out_specs=pl.BlockSpec((BATCH, BLK_V), lambda i: (0, i)),
        out_shape=jax.ShapeDtypeStruct((BATCH, VOCAB), jnp.float32),
        scratch_shapes=[pltpu.VMEM((BATCH, HIDDEN_DIM), jnp.float32)],
        compiler_params=pltpu.CompilerParams(
            dimension_semantics=("arbitrary",),
        ),
    )(embedded, W1, b1.reshape(1, HIDDEN_DIM), W2, b2.reshape(1, VOCAB))


def kernel(x, emb_table, W1, b1, W2, b2):
    embedded = _sc_gather(emb_table, x.astype(jnp.int32))
    return _tc_mlp(embedded, W1, b1, W2, b2)
